# baseline (speedup 1.0000x reference)
import functools

import jax
import jax.numpy as jnp
from jax import lax
from jax.experimental import pallas as pl
from jax.experimental.pallas import tpu as pltpu
from jax.experimental.pallas import tpu_sc as plsc

_T = 1000
_TPAD = 1024
_B = 16384
_NC = 2
_NS = 16
_NW = _NC * _NS
_BPW = _B // _NW
_L = 16


def _tables_body(betas_ref, out_ref):
    b = betas_ref[...]
    lane = lax.broadcasted_iota(jnp.int32, (1, _TPAD), 1)
    a = 1.0 - b
    ab = a
    s = 1
    while s < _TPAD:
        ab = ab * jnp.where(lane < s, 1.0, pltpu.roll(ab, s, 1))
        s *= 2
    ab_prev = jnp.where(lane < 1, 1.0, pltpu.roll(ab, 1, 1))
    bt = (1.0 - ab_prev) / (1.0 - ab) * b
    out_ref[0:1, :] = a
    out_ref[1:2, :] = ab
    out_ref[2:3, :] = b
    out_ref[3:4, :] = bt
    out_ref[4:5, :] = jnp.sqrt(b)


_tables = pl.pallas_call(
    _tables_body,
    out_shape=jax.ShapeDtypeStruct((5, _TPAD), jnp.float32),
)


@functools.cache
def _make_sc_gather():
    mesh = plsc.VectorSubcoreMesh(
        core_axis_name="c", subcore_axis_name="s",
        num_cores=_NC, num_subcores=_NS)

    @functools.partial(
        pl.kernel,
        out_type=jax.ShapeDtypeStruct((5, _B), jnp.float32),
        mesh=mesh,
        compiler_params=pltpu.CompilerParams(needs_layout_passes=False),
        scratch_types=[
            pltpu.VMEM((5, _TPAD), jnp.float32),
            pltpu.VMEM((_BPW,), jnp.int32),
            pltpu.VMEM((5, _BPW), jnp.float32),
        ],
    )
    def _sc_gather(tab_hbm, t_hbm, out_hbm, tab_v, idx_v, out_v):
        wid = lax.axis_index("s") * _NC + lax.axis_index("c")
        base = wid * _BPW
        pltpu.sync_copy(tab_hbm, tab_v)
        pltpu.sync_copy(t_hbm.at[pl.ds(base, _BPW)], idx_v)
        for i in range(_BPW // _L):
            idx = idx_v[pl.ds(i * _L, _L)]
            for j in range(5):
                row = jnp.full((_L,), j, jnp.int32)
                out_v[j, pl.ds(i * _L, _L)] = plsc.load_gather(
                    tab_v, [row, idx])
        pltpu.sync_copy(out_v, out_hbm.at[:, pl.ds(base, _BPW)])

    return _sc_gather


def kernel(t, betas):
    betas_pad = jnp.pad(betas.astype(jnp.float32),
                        (0, _TPAD - _T)).reshape(1, _TPAD)
    tables = _tables(betas_pad)
    return _make_sc_gather()(tables, t.astype(jnp.int32))


# speedup vs baseline: 1.0170x; 1.0170x over previous
import functools

import jax
import jax.numpy as jnp
from jax import lax
from jax.experimental import pallas as pl
from jax.experimental.pallas import tpu as pltpu
from jax.experimental.pallas import tpu_sc as plsc

_T = 1000
_TPAD = 1024
_B = 16384
_NC = 2
_NS = 16
_NW = _NC * _NS
_BPW = _B // _NW
_L = 16


def _tables_body(betas_ref, out_ref):
    b = jnp.concatenate(
        [betas_ref[...], jnp.zeros((1, _TPAD - _T), jnp.float32)], axis=1)
    lane = lax.broadcasted_iota(jnp.int32, (1, _TPAD), 1)
    a = 1.0 - b
    ab = a
    s = 1
    while s < _TPAD:
        ab = ab * jnp.where(lane < s, 1.0, pltpu.roll(ab, s, 1))
        s *= 2
    ab_prev = jnp.where(lane < 1, 1.0, pltpu.roll(ab, 1, 1))
    bt = (1.0 - ab_prev) / (1.0 - ab) * b
    out_ref[0:1, :] = a
    out_ref[1:2, :] = ab
    out_ref[2:3, :] = b
    out_ref[3:4, :] = bt
    out_ref[4:5, :] = jnp.sqrt(b)


_tables = pl.pallas_call(
    _tables_body,
    out_shape=jax.ShapeDtypeStruct((5, _TPAD), jnp.float32),
)


@functools.cache
def _make_sc_gather():
    mesh = plsc.VectorSubcoreMesh(
        core_axis_name="c", subcore_axis_name="s",
        num_cores=_NC, num_subcores=_NS)

    @functools.partial(
        pl.kernel,
        out_type=jax.ShapeDtypeStruct((5, _B), jnp.float32),
        mesh=mesh,
        compiler_params=pltpu.CompilerParams(needs_layout_passes=False),
        scratch_types=[
            pltpu.VMEM((5, _TPAD), jnp.float32),
            pltpu.VMEM((_BPW,), jnp.int32),
            pltpu.VMEM((5, _BPW), jnp.float32),
        ],
    )
    def _sc_gather(tab_hbm, t_hbm, out_hbm, tab_v, idx_v, out_v):
        wid = lax.axis_index("s") * _NC + lax.axis_index("c")
        base = wid * _BPW
        pltpu.sync_copy(tab_hbm, tab_v)
        pltpu.sync_copy(t_hbm.at[pl.ds(base, _BPW)], idx_v)
        for i in range(_BPW // _L):
            idx = idx_v[pl.ds(i * _L, _L)]
            for j in range(5):
                row = jnp.full((_L,), j, jnp.int32)
                out_v[j, pl.ds(i * _L, _L)] = plsc.load_gather(
                    tab_v, [row, idx])
        pltpu.sync_copy(out_v, out_hbm.at[:, pl.ds(base, _BPW)])

    return _sc_gather


def kernel(t, betas):
    tables = _tables(betas.astype(jnp.float32).reshape(1, _T))
    return _make_sc_gather()(tables, t.astype(jnp.int32))


# speedup vs baseline: 1.0409x; 1.0235x over previous
import functools

import jax
import jax.numpy as jnp
from jax import lax
from jax.experimental import pallas as pl
from jax.experimental.pallas import tpu as pltpu
from jax.experimental.pallas import tpu_sc as plsc

_T = 1000
_TPAD = 1024
_B = 16384
_NC = 2
_NS = 16
_NW = _NC * _NS
_BPW = _B // _NW
_L = 16


def _tables_body(betas_ref, out_ref):
    b = jnp.concatenate(
        [betas_ref[...], jnp.zeros((1, _TPAD - _T), jnp.float32)], axis=1)
    lane = lax.broadcasted_iota(jnp.int32, (1, _TPAD), 1)
    a = 1.0 - b
    ab = a
    s = 1
    while s < _TPAD:
        ab = ab * jnp.where(lane < s, 1.0, pltpu.roll(ab, s, 1))
        s *= 2
    ab_prev = jnp.where(lane < 1, 1.0, pltpu.roll(ab, 1, 1))
    bt = (1.0 - ab_prev) / (1.0 - ab) * b
    out_ref[0:1, :] = a
    out_ref[1:2, :] = ab
    out_ref[2:3, :] = b
    out_ref[3:4, :] = bt
    out_ref[4:5, :] = jnp.sqrt(b)


_tables = pl.pallas_call(
    _tables_body,
    out_shape=jax.ShapeDtypeStruct((5, _TPAD), jnp.float32),
)


@functools.cache
def _make_sc_gather():
    mesh = plsc.VectorSubcoreMesh(
        core_axis_name="c", subcore_axis_name="s",
        num_cores=_NC, num_subcores=_NS)

    @functools.partial(
        pl.kernel,
        out_type=jax.ShapeDtypeStruct((5, _B), jnp.float32),
        mesh=mesh,
        compiler_params=pltpu.CompilerParams(needs_layout_passes=False),
        scratch_types=[
            pltpu.VMEM((5, _TPAD), jnp.float32),
            pltpu.VMEM((_BPW,), jnp.int32),
            pltpu.VMEM((5, _BPW), jnp.float32),
            pltpu.SemaphoreType.DMA,
            pltpu.SemaphoreType.DMA,
        ],
    )
    def _sc_gather(tab_hbm, t_hbm, out_hbm, tab_v, idx_v, out_v,
                   sem_tab, sem_idx):
        wid = lax.axis_index("s") * _NC + lax.axis_index("c")
        base = wid * _BPW
        tab_cp = pltpu.async_copy(tab_hbm, tab_v, sem_tab)
        idx_cp = pltpu.async_copy(t_hbm.at[pl.ds(base, _BPW)], idx_v, sem_idx)
        tab_cp.wait()
        idx_cp.wait()
        for i in range(_BPW // _L):
            idx = idx_v[pl.ds(i * _L, _L)]
            for j in range(5):
                row = jnp.full((_L,), j, jnp.int32)
                out_v[j, pl.ds(i * _L, _L)] = plsc.load_gather(
                    tab_v, [row, idx])
        pltpu.sync_copy(out_v, out_hbm.at[:, pl.ds(base, _BPW)])

    return _sc_gather


def kernel(t, betas):
    tables = _tables(betas.astype(jnp.float32).reshape(1, _T))
    return _make_sc_gather()(tables, t.astype(jnp.int32))
